# baseline (speedup 1.0000x reference)
import jax
import jax.numpy as jnp
from jax import lax
from jax.experimental import pallas as pl
from jax.experimental.pallas import tpu as pltpu
from jax.experimental.pallas import tpu_sc as plsc

N_NODES = 10000
N_EDGES = 320000
LATENT = 128
HIDDEN = 256

NC = 2
NS = 16
NW = NC * NS

_EB = 128
_NBLK = N_EDGES // _EB
_STRIPE = 624
_ZR = 48

_NCH = 5
_CHB = _NBLK // _NCH
_ECH = N_EDGES // _NCH


def _sc_mesh():
    return plsc.VectorSubcoreMesh(core_axis_name="c", subcore_axis_name="s",
                                  num_cores=NC, num_subcores=NS)


_NSLOT = 2496 // NW


def _segsum_body(edges_hbm, w_hbm, dst_hbm, out_hbm, acc, ebuf, ibuf, wbuf,
                 ls0, ls1, ls2, ss0, ss1, ss2):
    c = lax.axis_index("c")
    s = lax.axis_index("s")
    wid = c * NS + s
    zero16 = jnp.zeros((16,), jnp.float32)
    ldsems = (ls0, ls1, ls2)
    scsems = (ss0, ss1, ss2)

    @pl.loop(0, 8)
    def _(r):
        for j in range(LATENT // 16):
            ebuf[0, r, pl.ds(j * 16, 16)] = zero16

    row0 = s * _STRIPE
    zsrc = ebuf.at[0].at[pl.ds(0, 8)]

    @pl.loop(0, _STRIPE, step=8)
    def _(r):
        pltpu.sync_copy(zsrc, acc.at[pl.ds(row0 + r, 8)])

    @pl.when(s == NS - 1)
    def _():
        pltpu.sync_copy(zsrc, acc.at[pl.ds(NS * _STRIPE, 8)])
        pltpu.sync_copy(zsrc, acc.at[pl.ds(NS * _STRIPE + 8, 8)])

    plsc.subcore_barrier()

    def issue_load(p, b):
        eb = b * _EB
        pltpu.async_copy(edges_hbm.at[pl.ds(eb, _EB)], ebuf.at[p], ldsems[p])
        pltpu.async_copy(w_hbm.at[pl.ds(eb, _EB)], wbuf.at[p], ldsems[p])
        pltpu.async_copy(dst_hbm.at[pl.ds(1, 1), pl.ds(eb, _EB)],
                         ibuf.at[p], ldsems[p])

    def wait_load(p):
        pltpu.make_async_copy(edges_hbm.at[pl.ds(0, _EB)], ebuf.at[p],
                              ldsems[p]).wait()
        pltpu.make_async_copy(w_hbm.at[pl.ds(0, _EB)], wbuf.at[p],
                              ldsems[p]).wait()
        pltpu.make_async_copy(dst_hbm.at[pl.ds(1, 1), pl.ds(0, _EB)],
                              ibuf.at[p], ldsems[p]).wait()

    def compute(p):
        ebufp = ebuf.at[p]
        wbufp = wbuf.at[p]

        @pl.loop(0, _EB, step=16)
        def _(e0):
            wv16 = wbufp[pl.ds(e0, 16)]
            for i in range(16):
                wv = jnp.full((16,), wv16[i], jnp.float32)
                for j in range(LATENT // 16):
                    sl = (e0 + i, pl.ds(j * 16, 16))
                    ebufp[sl] = ebufp[sl] * wv

    def issue_scatter(p):
        pltpu.async_copy(ebuf.at[p], acc.at[ibuf.at[p].at[0]], scsems[p],
                         add=True)

    def wait_scatter(p):
        pltpu.make_async_copy(ebuf.at[p], acc.at[ibuf.at[p].at[0]],
                              scsems[p]).wait()

    def slot(tt, p, q, first=False):
        wait_load(p)
        compute(p)
        if not first:
            wait_scatter(q)

        @pl.when(tt + 2 < _NSLOT)
        def _():
            issue_load(q, wid + NW * (tt + 2))

        issue_scatter(p)

    issue_load(0, wid)
    issue_load(1, wid + NW)
    slot(0, 0, 2, first=True)
    slot(1, 1, 0)
    slot(2, 2, 1)

    @pl.loop(3, _NSLOT, step=3)
    def _(t):
        slot(t, 0, 2)
        slot(t + 1, 1, 0)
        slot(t + 2, 2, 1)

    wait_scatter((_NSLOT - 1) % 3)

    @pl.when(wid < _NBLK - _NSLOT * NW)
    def _():
        b = _NSLOT * NW + wid
        eb = b * _EB
        pltpu.sync_copy(edges_hbm.at[pl.ds(eb, _EB)], ebuf.at[0])
        pltpu.sync_copy(w_hbm.at[pl.ds(eb, _EB)], wbuf.at[0])
        pltpu.sync_copy(dst_hbm.at[pl.ds(1, 1), pl.ds(eb, _EB)], ibuf.at[0])
        compute(0)
        pltpu.sync_copy(ebuf.at[0], acc.at[ibuf.at[0].at[0]], add=True)

    plsc.subcore_barrier()
    pltpu.sync_copy(acc.at[pl.ds(row0, _STRIPE)],
                    out_hbm.at[c].at[pl.ds(row0, _STRIPE)])

    @pl.when(s == NS - 1)
    def _():
        pltpu.sync_copy(acc.at[pl.ds(NS * _STRIPE, 16)],
                        out_hbm.at[c].at[pl.ds(NS * _STRIPE, 16)])


def _sc_segsum(edges, w, dst2d):
    k = pl.kernel(
        _segsum_body,
        out_type=jax.ShapeDtypeStruct((NC, N_NODES, LATENT), jnp.float32),
        mesh=_sc_mesh(),
        scratch_types=[
            pltpu.VMEM_SHARED((N_NODES, LATENT), jnp.float32),
            pltpu.VMEM((3, _EB, LATENT), jnp.float32),
            pltpu.VMEM((3, 1, _EB), jnp.int32),
            pltpu.VMEM((3, _EB), jnp.float32),
            pltpu.SemaphoreType.DMA,
            pltpu.SemaphoreType.DMA,
            pltpu.SemaphoreType.DMA,
            pltpu.SemaphoreType.DMA,
            pltpu.SemaphoreType.DMA,
            pltpu.SemaphoreType.DMA,
        ],
    )
    return k(edges, w, dst2d)


def _make_gather_chunk(chunk):
    b_lo = chunk * _CHB
    b_hi = b_lo + _CHB

    def body(n_hbm, g_hbm, g1_hbm, g2_hbm, ibuf, rbuf):
        c = lax.axis_index("c")
        s = lax.axis_index("s")
        wid = c * NS + s

        for which, out in ((0, g1_hbm), (1, g2_hbm)):
            @pl.loop(b_lo + wid, b_hi, step=NW)
            def _(b):
                eb = b * _EB
                pltpu.sync_copy(g_hbm.at[pl.ds(which, 1), pl.ds(eb, _EB)], ibuf)
                pltpu.sync_copy(n_hbm.at[ibuf.at[0]], rbuf)
                pltpu.sync_copy(rbuf, out.at[pl.ds(eb - b_lo * _EB, _EB)])

    return pl.kernel(
        body,
        out_type=(jax.ShapeDtypeStruct((_ECH, LATENT), jnp.float32),
                  jax.ShapeDtypeStruct((_ECH, LATENT), jnp.float32)),
        mesh=_sc_mesh(),
        scratch_types=[
            pltpu.VMEM((1, _EB), jnp.int32),
            pltpu.VMEM((_EB, LATENT), jnp.float32),
        ],
    )


def _ln(x, g, b):
    n = x.shape[-1]
    ones = jnp.ones((n, 2), jnp.bfloat16)
    xb = x.astype(jnp.bfloat16)
    s1 = jnp.dot(xb, ones, preferred_element_type=jnp.float32)[:, :1]
    s2 = jnp.dot(xb * xb, ones, preferred_element_type=jnp.float32)[:, :1]
    m = s1 * (1.0 / n)
    v = s2 * (1.0 / n) - m * m
    inv = lax.rsqrt(v + 1e-5)
    return (x * inv - m * inv) * g + b


def _dot(a, b):
    return jnp.dot(a, b, preferred_element_type=jnp.float32)


def _b16(x):
    return x.astype(jnp.bfloat16)


def _node_body(x_ref, m0_ref, m1_ref,
               W0a, W0b, b0, g0, be0, W1, b1, g1, be1, W2, b2, g2, be2,
               out_ref):
    x = x_ref[...]
    m = m0_ref[...] + m1_ref[...]
    h = _dot(_b16(x), W0a[...]) + _dot(_b16(m), W0b[...]) + b0[...]
    h = jax.nn.relu(_ln(h, g0[...], be0[...]))
    h = _dot(_b16(h), W1[...]) + b1[...]
    h = jax.nn.relu(_ln(h, g1[...], be1[...]))
    h = _dot(_b16(h), W2[...]) + b2[...]
    h = jax.nn.relu(_ln(h, g2[...], be2[...]))
    out_ref[...] = h + x


_NB = 2000


def _tc_node_mlp(nodes, m0, m1, params):
    grid = (N_NODES // _NB,)
    row_spec = pl.BlockSpec((_NB, LATENT), lambda i: (i, 0))

    def w_spec(a):
        return pl.BlockSpec(a.shape, lambda i: tuple(0 for _ in a.shape))

    return pl.pallas_call(
        _node_body,
        grid=grid,
        in_specs=[row_spec, row_spec, row_spec] + [w_spec(a) for a in params],
        out_specs=row_spec,
        out_shape=jax.ShapeDtypeStruct((N_NODES, LATENT), jnp.float32),
    )(nodes, m0, m1, *params)


def _edge_body(e_ref, g1_ref, g2_ref,
               W0a, W0b, W0c, b0, g0, be0, W1, b1, g1, be1, W2, b2, g2, be2,
               out_ref):
    e = e_ref[...]
    h = (_dot(_b16(g1_ref[...]), W0a[...]) + _dot(_b16(g2_ref[...]), W0b[...])
         + _dot(_b16(e), W0c[...]) + b0[...])
    h = jax.nn.relu(_ln(h, g0[...], be0[...]))
    h = _dot(_b16(h), W1[...]) + b1[...]
    h = jax.nn.relu(_ln(h, g1[...], be1[...]))
    h = _dot(_b16(h), W2[...]) + b2[...]
    h = jnp.tanh(_ln(h, g2[...], be2[...]))
    out_ref[...] = h + e


_EBK = 1600


def _tc_edge_mlp_chunk(buf, edges, gs, gd, params, chunk):
    grid = (_ECH // _EBK,)
    blk0 = chunk * (_ECH // _EBK)
    full_spec = pl.BlockSpec((_EBK, LATENT), lambda i: (blk0 + i, 0))
    ch_spec = pl.BlockSpec((_EBK, LATENT), lambda i: (i, 0))

    def w_spec(a):
        return pl.BlockSpec(a.shape, lambda i: tuple(0 for _ in a.shape))

    out_shape = jax.ShapeDtypeStruct((N_EDGES, LATENT), jnp.float32)
    if buf is None:
        return pl.pallas_call(
            _edge_body,
            grid=grid,
            in_specs=[full_spec, ch_spec, ch_spec] + [w_spec(a) for a in params],
            out_specs=full_spec,
            out_shape=out_shape,
        )(edges, gs, gd, *params)

    dummy_spec = pl.BlockSpec((8, LATENT), lambda i: (0, 0))
    return pl.pallas_call(
        lambda dummy_ref, *rest: _edge_body(*rest),
        grid=grid,
        in_specs=[dummy_spec, full_spec, ch_spec, ch_spec]
                 + [w_spec(a) for a in params],
        out_specs=full_spec,
        out_shape=out_shape,
        input_output_aliases={0: 0},
    )(buf, edges, gs, gd, *params)


def kernel(nodes, edges, graph, edge_weights,
           node_W0, node_b0, node_g0, node_be0,
           node_W1, node_b1, node_g1, node_be1,
           node_W2, node_b2, node_g2, node_be2,
           edge_W0, edge_b0, edge_g0, edge_be0,
           edge_W1, edge_b1, edge_g1, edge_be1,
           edge_W2, edge_b2, edge_g2, edge_be2):
    w = edge_weights.reshape(N_EDGES)

    partials = _sc_segsum(edges, w, graph)

    r2 = lambda a: a.reshape(1, -1)
    b16 = lambda a: a.astype(jnp.bfloat16)
    node_params = (b16(node_W0[:LATENT]), b16(node_W0[LATENT:]), r2(node_b0),
                   r2(node_g0), r2(node_be0),
                   b16(node_W1), r2(node_b1), r2(node_g1), r2(node_be1),
                   b16(node_W2), r2(node_b2), r2(node_g2), r2(node_be2))
    nodes2 = _tc_node_mlp(nodes, partials[0], partials[1], node_params)

    edge_params = (b16(edge_W0[:LATENT]), b16(edge_W0[LATENT:2 * LATENT]),
                   b16(edge_W0[2 * LATENT:]), r2(edge_b0),
                   r2(edge_g0), r2(edge_be0),
                   b16(edge_W1), r2(edge_b1), r2(edge_g1), r2(edge_be1),
                   b16(edge_W2), r2(edge_b2), r2(edge_g2), r2(edge_be2))

    buf = None
    for k in range(_NCH):
        gs, gd = _make_gather_chunk(k)(nodes2, graph)
        buf = _tc_edge_mlp_chunk(buf, edges, gs, gd, edge_params, k)

    return (nodes2, buf)


# speedup vs baseline: 1.0918x; 1.0918x over previous
import jax
import jax.numpy as jnp
from jax import lax
from jax.experimental import pallas as pl
from jax.experimental.pallas import tpu as pltpu
from jax.experimental.pallas import tpu_sc as plsc

N_NODES = 10000
N_EDGES = 320000
LATENT = 128
HIDDEN = 256

NC = 2
NS = 16
NW = NC * NS

_EB = 128
_NBLK = N_EDGES // _EB
_STRIPE = 624
_ZR = 48

_NCH = 5
_CHB = _NBLK // _NCH
_ECH = N_EDGES // _NCH


def _sc_mesh():
    return plsc.VectorSubcoreMesh(core_axis_name="c", subcore_axis_name="s",
                                  num_cores=NC, num_subcores=NS)


_NSLOT = 2496 // NW


def _segsum_body(edges_hbm, w_hbm, dst_hbm, out_hbm, acc, ebuf, ibuf, wbuf,
                 ls0, ls1, ls2, ss0, ss1, ss2):
    c = lax.axis_index("c")
    s = lax.axis_index("s")
    wid = c * NS + s
    zero16 = jnp.zeros((16,), jnp.float32)
    ldsems = (ls0, ls1, ls2)
    scsems = (ss0, ss1, ss2)

    @pl.loop(0, 8)
    def _(r):
        for j in range(LATENT // 16):
            ebuf[0, r, pl.ds(j * 16, 16)] = zero16

    row0 = s * _STRIPE
    zsrc = ebuf.at[0].at[pl.ds(0, 8)]

    @pl.loop(0, _STRIPE, step=8)
    def _(r):
        pltpu.sync_copy(zsrc, acc.at[pl.ds(row0 + r, 8)])

    @pl.when(s == NS - 1)
    def _():
        pltpu.sync_copy(zsrc, acc.at[pl.ds(NS * _STRIPE, 8)])
        pltpu.sync_copy(zsrc, acc.at[pl.ds(NS * _STRIPE + 8, 8)])

    plsc.subcore_barrier()

    def issue_load(p, b):
        eb = b * _EB
        pltpu.async_copy(edges_hbm.at[pl.ds(eb, _EB)], ebuf.at[p], ldsems[p])
        pltpu.async_copy(w_hbm.at[pl.ds(eb, _EB)], wbuf.at[p], ldsems[p])
        pltpu.async_copy(dst_hbm.at[pl.ds(1, 1), pl.ds(eb, _EB)],
                         ibuf.at[p], ldsems[p])

    def wait_load(p):
        pltpu.make_async_copy(edges_hbm.at[pl.ds(0, _EB)], ebuf.at[p],
                              ldsems[p]).wait()
        pltpu.make_async_copy(w_hbm.at[pl.ds(0, _EB)], wbuf.at[p],
                              ldsems[p]).wait()
        pltpu.make_async_copy(dst_hbm.at[pl.ds(1, 1), pl.ds(0, _EB)],
                              ibuf.at[p], ldsems[p]).wait()

    def compute(p):
        ebufp = ebuf.at[p]
        wbufp = wbuf.at[p]

        @pl.loop(0, _EB, step=16)
        def _(e0):
            wv16 = wbufp[pl.ds(e0, 16)]
            for i in range(16):
                wv = jnp.full((16,), wv16[i], jnp.float32)
                for j in range(LATENT // 16):
                    sl = (e0 + i, pl.ds(j * 16, 16))
                    ebufp[sl] = ebufp[sl] * wv

    def issue_scatter(p):
        pltpu.async_copy(ebuf.at[p], acc.at[ibuf.at[p].at[0]], scsems[p],
                         add=True)

    def wait_scatter(p):
        pltpu.make_async_copy(ebuf.at[p], acc.at[ibuf.at[p].at[0]],
                              scsems[p]).wait()

    def slot(tt, p, q, first=False):
        wait_load(p)
        compute(p)
        if not first:
            wait_scatter(q)

        @pl.when(tt + 2 < _NSLOT)
        def _():
            issue_load(q, wid + NW * (tt + 2))

        issue_scatter(p)

    issue_load(0, wid)
    issue_load(1, wid + NW)
    slot(0, 0, 2, first=True)
    slot(1, 1, 0)
    slot(2, 2, 1)

    @pl.loop(3, _NSLOT, step=3)
    def _(t):
        slot(t, 0, 2)
        slot(t + 1, 1, 0)
        slot(t + 2, 2, 1)

    wait_scatter((_NSLOT - 1) % 3)

    @pl.when(wid < _NBLK - _NSLOT * NW)
    def _():
        b = _NSLOT * NW + wid
        eb = b * _EB
        pltpu.sync_copy(edges_hbm.at[pl.ds(eb, _EB)], ebuf.at[0])
        pltpu.sync_copy(w_hbm.at[pl.ds(eb, _EB)], wbuf.at[0])
        pltpu.sync_copy(dst_hbm.at[pl.ds(1, 1), pl.ds(eb, _EB)], ibuf.at[0])
        compute(0)
        pltpu.sync_copy(ebuf.at[0], acc.at[ibuf.at[0].at[0]], add=True)

    plsc.subcore_barrier()
    pltpu.sync_copy(acc.at[pl.ds(row0, _STRIPE)],
                    out_hbm.at[c].at[pl.ds(row0, _STRIPE)])

    @pl.when(s == NS - 1)
    def _():
        pltpu.sync_copy(acc.at[pl.ds(NS * _STRIPE, 16)],
                        out_hbm.at[c].at[pl.ds(NS * _STRIPE, 16)])


def _sc_segsum(edges, w, dst2d):
    k = pl.kernel(
        _segsum_body,
        out_type=jax.ShapeDtypeStruct((NC, N_NODES, LATENT), jnp.float32),
        mesh=_sc_mesh(),
        scratch_types=[
            pltpu.VMEM_SHARED((N_NODES, LATENT), jnp.float32),
            pltpu.VMEM((3, _EB, LATENT), jnp.float32),
            pltpu.VMEM((3, 1, _EB), jnp.int32),
            pltpu.VMEM((3, _EB), jnp.float32),
            pltpu.SemaphoreType.DMA,
            pltpu.SemaphoreType.DMA,
            pltpu.SemaphoreType.DMA,
            pltpu.SemaphoreType.DMA,
            pltpu.SemaphoreType.DMA,
            pltpu.SemaphoreType.DMA,
        ],
    )
    return k(edges, w, dst2d)


def _make_gather_chunk(chunk):
    b_lo = chunk * _CHB
    b_hi = b_lo + _CHB

    def body(n_hbm, g_hbm, g1_hbm, g2_hbm, ibuf, rbuf):
        c = lax.axis_index("c")
        s = lax.axis_index("s")
        wid = c * NS + s

        for which, out in ((0, g1_hbm), (1, g2_hbm)):
            @pl.loop(b_lo + wid, b_hi, step=NW)
            def _(b):
                eb = b * _EB
                pltpu.sync_copy(g_hbm.at[pl.ds(which, 1), pl.ds(eb, _EB)], ibuf)
                pltpu.sync_copy(n_hbm.at[ibuf.at[0]], rbuf)
                pltpu.sync_copy(rbuf, out.at[pl.ds(eb - b_lo * _EB, _EB)])

    return pl.kernel(
        body,
        out_type=(jax.ShapeDtypeStruct((_ECH, LATENT), jnp.float32),
                  jax.ShapeDtypeStruct((_ECH, LATENT), jnp.float32)),
        mesh=_sc_mesh(),
        scratch_types=[
            pltpu.VMEM((1, _EB), jnp.int32),
            pltpu.VMEM((_EB, LATENT), jnp.float32),
        ],
    )


def _ln(x, g, b):
    n = x.shape[-1]
    ones = jnp.ones((n, 2), jnp.bfloat16)
    xb = x.astype(jnp.bfloat16)
    s1 = jnp.dot(xb, ones, preferred_element_type=jnp.float32)[:, :1]
    s2 = jnp.dot(xb * xb, ones, preferred_element_type=jnp.float32)[:, :1]
    m = s1 * (1.0 / n)
    v = s2 * (1.0 / n) - m * m
    inv = lax.rsqrt(v + 1e-5)
    return (x * inv - m * inv) * g + b


def _dot(a, b):
    return jnp.dot(a, b, preferred_element_type=jnp.float32)


def _b16(x):
    return x.astype(jnp.bfloat16)


def _node_body(x_ref, m0_ref, m1_ref,
               W0a, W0b, b0, g0, be0, W1, b1, g1, be1, W2, b2, g2, be2,
               out_ref):
    x = x_ref[...]
    m = m0_ref[...] + m1_ref[...]
    h = _dot(_b16(x), W0a[...]) + _dot(_b16(m), W0b[...]) + b0[...]
    h = jax.nn.relu(_ln(h, g0[...], be0[...]))
    h = _dot(_b16(h), W1[...]) + b1[...]
    h = jax.nn.relu(_ln(h, g1[...], be1[...]))
    h = _dot(_b16(h), W2[...]) + b2[...]
    h = jax.nn.relu(_ln(h, g2[...], be2[...]))
    out_ref[...] = h + x


_NB = 5000


def _tc_node_mlp(nodes, m0, m1, params):
    grid = (N_NODES // _NB,)
    row_spec = pl.BlockSpec((_NB, LATENT), lambda i: (i, 0))

    def w_spec(a):
        return pl.BlockSpec(a.shape, lambda i: tuple(0 for _ in a.shape))

    return pl.pallas_call(
        _node_body,
        grid=grid,
        in_specs=[row_spec, row_spec, row_spec] + [w_spec(a) for a in params],
        out_specs=row_spec,
        out_shape=jax.ShapeDtypeStruct((N_NODES, LATENT), jnp.float32),
    )(nodes, m0, m1, *params)


def _edge_body(e_ref, g1_ref, g2_ref,
               W0a, W0b, W0c, b0, g0, be0, W1, b1, g1, be1, W2, b2, g2, be2,
               out_ref):
    e = e_ref[...]
    h = (_dot(_b16(g1_ref[...]), W0a[...]) + _dot(_b16(g2_ref[...]), W0b[...])
         + _dot(_b16(e), W0c[...]) + b0[...])
    h = jax.nn.relu(_ln(h, g0[...], be0[...]))
    h = _dot(_b16(h), W1[...]) + b1[...]
    h = jax.nn.relu(_ln(h, g1[...], be1[...]))
    h = _dot(_b16(h), W2[...]) + b2[...]
    h = jnp.tanh(_ln(h, g2[...], be2[...]))
    out_ref[...] = h + e


_EBK = 2000


def _tc_edge_mlp_chunk(buf, edges, gs, gd, params, chunk):
    grid = (_ECH // _EBK,)
    blk0 = chunk * (_ECH // _EBK)
    full_spec = pl.BlockSpec((_EBK, LATENT), lambda i: (blk0 + i, 0))
    ch_spec = pl.BlockSpec((_EBK, LATENT), lambda i: (i, 0))

    def w_spec(a):
        return pl.BlockSpec(a.shape, lambda i: tuple(0 for _ in a.shape))

    out_shape = jax.ShapeDtypeStruct((N_EDGES, LATENT), jnp.float32)
    if buf is None:
        return pl.pallas_call(
            _edge_body,
            grid=grid,
            in_specs=[full_spec, ch_spec, ch_spec] + [w_spec(a) for a in params],
            out_specs=full_spec,
            out_shape=out_shape,
        )(edges, gs, gd, *params)

    dummy_spec = pl.BlockSpec((8, LATENT), lambda i: (0, 0))
    return pl.pallas_call(
        lambda dummy_ref, *rest: _edge_body(*rest),
        grid=grid,
        in_specs=[dummy_spec, full_spec, ch_spec, ch_spec]
                 + [w_spec(a) for a in params],
        out_specs=full_spec,
        out_shape=out_shape,
        input_output_aliases={0: 0},
    )(buf, edges, gs, gd, *params)


def kernel(nodes, edges, graph, edge_weights,
           node_W0, node_b0, node_g0, node_be0,
           node_W1, node_b1, node_g1, node_be1,
           node_W2, node_b2, node_g2, node_be2,
           edge_W0, edge_b0, edge_g0, edge_be0,
           edge_W1, edge_b1, edge_g1, edge_be1,
           edge_W2, edge_b2, edge_g2, edge_be2):
    w = edge_weights.reshape(N_EDGES)

    partials = _sc_segsum(edges, w, graph)

    r2 = lambda a: a.reshape(1, -1)
    b16 = lambda a: a.astype(jnp.bfloat16)
    node_params = (b16(node_W0[:LATENT]), b16(node_W0[LATENT:]), r2(node_b0),
                   r2(node_g0), r2(node_be0),
                   b16(node_W1), r2(node_b1), r2(node_g1), r2(node_be1),
                   b16(node_W2), r2(node_b2), r2(node_g2), r2(node_be2))
    nodes2 = _tc_node_mlp(nodes, partials[0], partials[1], node_params)

    edge_params = (b16(edge_W0[:LATENT]), b16(edge_W0[LATENT:2 * LATENT]),
                   b16(edge_W0[2 * LATENT:]), r2(edge_b0),
                   r2(edge_g0), r2(edge_be0),
                   b16(edge_W1), r2(edge_b1), r2(edge_g1), r2(edge_be1),
                   b16(edge_W2), r2(edge_b2), r2(edge_g2), r2(edge_be2))

    buf = None
    for k in range(_NCH):
        gs, gd = _make_gather_chunk(k)(nodes2, graph)
        buf = _tc_edge_mlp_chunk(buf, edges, gs, gd, edge_params, k)

    return (nodes2, buf)


# speedup vs baseline: 1.3024x; 1.1929x over previous
import jax
import jax.numpy as jnp
from jax import lax
from jax.experimental import pallas as pl
from jax.experimental.pallas import tpu as pltpu
from jax.experimental.pallas import tpu_sc as plsc

N_NODES = 10000
N_EDGES = 320000
LATENT = 128
HIDDEN = 256

NC = 2
NS = 16
NW = NC * NS

_EB = 128
_NBLK = N_EDGES // _EB
_STRIPE = 624
_ZR = 48

_NCH = 5
_CHB = _NBLK // _NCH
_ECH = N_EDGES // _NCH


def _sc_mesh():
    return plsc.VectorSubcoreMesh(core_axis_name="c", subcore_axis_name="s",
                                  num_cores=NC, num_subcores=NS)


_NSLOT = 2496 // NW


def _segsum_body(edges_hbm, w_hbm, dst_hbm, out_hbm, acc, ebuf, ibuf, wbuf,
                 ls0, ls1, ls2, ss0, ss1, ss2):
    c = lax.axis_index("c")
    s = lax.axis_index("s")
    wid = c * NS + s
    zero16 = jnp.zeros((16,), jnp.float32)
    ldsems = (ls0, ls1, ls2)
    scsems = (ss0, ss1, ss2)

    @pl.loop(0, 8)
    def _(r):
        for j in range(LATENT // 16):
            ebuf[0, r, pl.ds(j * 16, 16)] = zero16

    row0 = s * _STRIPE
    zsrc = ebuf.at[0].at[pl.ds(0, 8)]

    @pl.loop(0, _STRIPE, step=8)
    def _(r):
        pltpu.sync_copy(zsrc, acc.at[pl.ds(row0 + r, 8)])

    @pl.when(s == NS - 1)
    def _():
        pltpu.sync_copy(zsrc, acc.at[pl.ds(NS * _STRIPE, 8)])
        pltpu.sync_copy(zsrc, acc.at[pl.ds(NS * _STRIPE + 8, 8)])

    plsc.subcore_barrier()

    def issue_load(p, b):
        eb = b * _EB
        pltpu.async_copy(edges_hbm.at[pl.ds(eb, _EB)], ebuf.at[p], ldsems[p])
        pltpu.async_copy(w_hbm.at[pl.ds(eb, _EB)], wbuf.at[p], ldsems[p])
        pltpu.async_copy(dst_hbm.at[pl.ds(1, 1), pl.ds(eb, _EB)],
                         ibuf.at[p], ldsems[p])

    def wait_load(p):
        pltpu.make_async_copy(edges_hbm.at[pl.ds(0, _EB)], ebuf.at[p],
                              ldsems[p]).wait()
        pltpu.make_async_copy(w_hbm.at[pl.ds(0, _EB)], wbuf.at[p],
                              ldsems[p]).wait()
        pltpu.make_async_copy(dst_hbm.at[pl.ds(1, 1), pl.ds(0, _EB)],
                              ibuf.at[p], ldsems[p]).wait()

    def compute(p):
        ebufp = ebuf.at[p]
        wbufp = wbuf.at[p]

        @pl.loop(0, _EB, step=16)
        def _(e0):
            wv16 = wbufp[pl.ds(e0, 16)]
            for i in range(16):
                wv = jnp.full((16,), wv16[i], jnp.float32)
                for j in range(LATENT // 16):
                    sl = (e0 + i, pl.ds(j * 16, 16))
                    ebufp[sl] = ebufp[sl] * wv

    def issue_scatter(p):
        pltpu.async_copy(ebuf.at[p], acc.at[ibuf.at[p].at[0]], scsems[p],
                         add=True)

    def wait_scatter(p):
        pltpu.make_async_copy(ebuf.at[p], acc.at[ibuf.at[p].at[0]],
                              scsems[p]).wait()

    def slot(tt, p, q, first=False):
        wait_load(p)
        compute(p)
        if not first:
            wait_scatter(q)

        @pl.when(tt + 2 < _NSLOT)
        def _():
            issue_load(q, wid + NW * (tt + 2))

        issue_scatter(p)

    issue_load(0, wid)
    issue_load(1, wid + NW)
    slot(0, 0, 2, first=True)
    slot(1, 1, 0)
    slot(2, 2, 1)

    @pl.loop(3, _NSLOT, step=3)
    def _(t):
        slot(t, 0, 2)
        slot(t + 1, 1, 0)
        slot(t + 2, 2, 1)

    wait_scatter((_NSLOT - 1) % 3)

    @pl.when(wid < _NBLK - _NSLOT * NW)
    def _():
        b = _NSLOT * NW + wid
        eb = b * _EB
        pltpu.sync_copy(edges_hbm.at[pl.ds(eb, _EB)], ebuf.at[0])
        pltpu.sync_copy(w_hbm.at[pl.ds(eb, _EB)], wbuf.at[0])
        pltpu.sync_copy(dst_hbm.at[pl.ds(1, 1), pl.ds(eb, _EB)], ibuf.at[0])
        compute(0)
        pltpu.sync_copy(ebuf.at[0], acc.at[ibuf.at[0].at[0]], add=True)

    plsc.subcore_barrier()
    pltpu.sync_copy(acc.at[pl.ds(row0, _STRIPE)],
                    out_hbm.at[c].at[pl.ds(row0, _STRIPE)])

    @pl.when(s == NS - 1)
    def _():
        pltpu.sync_copy(acc.at[pl.ds(NS * _STRIPE, 16)],
                        out_hbm.at[c].at[pl.ds(NS * _STRIPE, 16)])


def _sc_segsum(edges, w, dst2d):
    k = pl.kernel(
        _segsum_body,
        out_type=jax.ShapeDtypeStruct((NC, N_NODES, LATENT), jnp.float32),
        mesh=_sc_mesh(),
        scratch_types=[
            pltpu.VMEM_SHARED((N_NODES, LATENT), jnp.float32),
            pltpu.VMEM((3, _EB, LATENT), jnp.float32),
            pltpu.VMEM((3, 1, _EB), jnp.int32),
            pltpu.VMEM((3, _EB), jnp.float32),
            pltpu.SemaphoreType.DMA,
            pltpu.SemaphoreType.DMA,
            pltpu.SemaphoreType.DMA,
            pltpu.SemaphoreType.DMA,
            pltpu.SemaphoreType.DMA,
            pltpu.SemaphoreType.DMA,
        ],
    )
    return k(edges, w, dst2d)


def _make_gather_chunk(chunk):
    b_lo = chunk * _CHB
    b_hi = b_lo + _CHB

    def body(n_hbm, g_hbm, g1_hbm, g2_hbm, ibuf, rbuf):
        c = lax.axis_index("c")
        s = lax.axis_index("s")
        wid = c * NS + s

        for which, out in ((0, g1_hbm), (1, g2_hbm)):
            @pl.loop(b_lo + wid, b_hi, step=NW)
            def _(b):
                eb = b * _EB
                pltpu.sync_copy(g_hbm.at[pl.ds(which, 1), pl.ds(eb, _EB)], ibuf)
                pltpu.sync_copy(n_hbm.at[ibuf.at[0]], rbuf)
                pltpu.sync_copy(rbuf, out.at[pl.ds(eb - b_lo * _EB, _EB)])

    return pl.kernel(
        body,
        out_type=(jax.ShapeDtypeStruct((_ECH, LATENT), jnp.float32),
                  jax.ShapeDtypeStruct((_ECH, LATENT), jnp.float32)),
        mesh=_sc_mesh(),
        scratch_types=[
            pltpu.VMEM((1, _EB), jnp.int32),
            pltpu.VMEM((_EB, LATENT), jnp.float32),
        ],
    )


def _ln(x, g, b):
    m = jnp.mean(x, axis=-1, keepdims=True)
    xc = x - m
    v = jnp.mean(xc * xc, axis=-1, keepdims=True)
    inv = lax.rsqrt(v + 1e-5)
    return xc * inv * g + b


def _dot(a, b):
    return jnp.dot(a, b, preferred_element_type=jnp.float32)


def _b16(x):
    return x.astype(jnp.bfloat16)


def _node_body(x_ref, m0_ref, m1_ref,
               W0a, W0b, b0, g0, be0, W1, b1, g1, be1, W2, b2, g2, be2,
               out_ref):
    x = x_ref[...]
    m = m0_ref[...] + m1_ref[...]
    h = _dot(_b16(x), W0a[...]) + _dot(_b16(m), W0b[...]) + b0[...]
    h = jax.nn.relu(_ln(h, g0[...], be0[...]))
    h = _dot(_b16(h), W1[...]) + b1[...]
    h = jax.nn.relu(_ln(h, g1[...], be1[...]))
    h = _dot(_b16(h), W2[...]) + b2[...]
    h = jax.nn.relu(_ln(h, g2[...], be2[...]))
    out_ref[...] = h + x


_NB = 5000


def _tc_node_mlp(nodes, m0, m1, params):
    grid = (N_NODES // _NB,)
    row_spec = pl.BlockSpec((_NB, LATENT), lambda i: (i, 0))

    def w_spec(a):
        return pl.BlockSpec(a.shape, lambda i: tuple(0 for _ in a.shape))

    return pl.pallas_call(
        _node_body,
        grid=grid,
        in_specs=[row_spec, row_spec, row_spec] + [w_spec(a) for a in params],
        out_specs=row_spec,
        out_shape=jax.ShapeDtypeStruct((N_NODES, LATENT), jnp.float32),
    )(nodes, m0, m1, *params)


def _edge_body(e_ref, g1_ref, g2_ref,
               W0a, W0b, W0c, b0, g0, be0, W1, b1, g1, be1, W2, b2, g2, be2,
               out_ref):
    e = e_ref[...]
    h = (_dot(_b16(g1_ref[...]), W0a[...]) + _dot(_b16(g2_ref[...]), W0b[...])
         + _dot(_b16(e), W0c[...]) + b0[...])
    h = jax.nn.relu(_ln(h, g0[...], be0[...]))
    h = _dot(_b16(h), W1[...]) + b1[...]
    h = jax.nn.relu(_ln(h, g1[...], be1[...]))
    h = _dot(_b16(h), W2[...]) + b2[...]
    h = jnp.tanh(_ln(h, g2[...], be2[...]))
    out_ref[...] = h + e


_EBK = 2000


def _tc_edge_mlp_chunk(buf, edges, gs, gd, params, chunk):
    grid = (_ECH // _EBK,)
    blk0 = chunk * (_ECH // _EBK)
    full_spec = pl.BlockSpec((_EBK, LATENT), lambda i: (blk0 + i, 0))
    ch_spec = pl.BlockSpec((_EBK, LATENT), lambda i: (i, 0))

    def w_spec(a):
        return pl.BlockSpec(a.shape, lambda i: tuple(0 for _ in a.shape))

    out_shape = jax.ShapeDtypeStruct((N_EDGES, LATENT), jnp.float32)
    if buf is None:
        return pl.pallas_call(
            _edge_body,
            grid=grid,
            in_specs=[full_spec, ch_spec, ch_spec] + [w_spec(a) for a in params],
            out_specs=full_spec,
            out_shape=out_shape,
        )(edges, gs, gd, *params)

    dummy_spec = pl.BlockSpec((8, LATENT), lambda i: (0, 0))
    return pl.pallas_call(
        lambda dummy_ref, *rest: _edge_body(*rest),
        grid=grid,
        in_specs=[dummy_spec, full_spec, ch_spec, ch_spec]
                 + [w_spec(a) for a in params],
        out_specs=full_spec,
        out_shape=out_shape,
        input_output_aliases={0: 0},
    )(buf, edges, gs, gd, *params)


def kernel(nodes, edges, graph, edge_weights,
           node_W0, node_b0, node_g0, node_be0,
           node_W1, node_b1, node_g1, node_be1,
           node_W2, node_b2, node_g2, node_be2,
           edge_W0, edge_b0, edge_g0, edge_be0,
           edge_W1, edge_b1, edge_g1, edge_be1,
           edge_W2, edge_b2, edge_g2, edge_be2):
    w = edge_weights.reshape(N_EDGES)

    partials = _sc_segsum(edges, w, graph)

    r2 = lambda a: a.reshape(1, -1)
    b16 = lambda a: a.astype(jnp.bfloat16)
    node_params = (b16(node_W0[:LATENT]), b16(node_W0[LATENT:]), r2(node_b0),
                   r2(node_g0), r2(node_be0),
                   b16(node_W1), r2(node_b1), r2(node_g1), r2(node_be1),
                   b16(node_W2), r2(node_b2), r2(node_g2), r2(node_be2))
    nodes2 = _tc_node_mlp(nodes, partials[0], partials[1], node_params)

    edge_params = (b16(edge_W0[:LATENT]), b16(edge_W0[LATENT:2 * LATENT]),
                   b16(edge_W0[2 * LATENT:]), r2(edge_b0),
                   r2(edge_g0), r2(edge_be0),
                   b16(edge_W1), r2(edge_b1), r2(edge_g1), r2(edge_be1),
                   b16(edge_W2), r2(edge_b2), r2(edge_g2), r2(edge_be2))

    buf = None
    for k in range(_NCH):
        gs, gd = _make_gather_chunk(k)(nodes2, graph)
        buf = _tc_edge_mlp_chunk(buf, edges, gs, gd, edge_params, k)

    return (nodes2, buf)
